# Initial kernel scaffold; baseline (speedup 1.0000x reference)
import jax
import jax.numpy as jnp
from jax.experimental import pallas as pl


def kernel(out_cls, out_ellipse, anchors):
    raise NotImplementedError("write your pallas kernel here")


# speedup vs baseline: 15.2298x; 15.2298x over previous
import jax
import jax.numpy as jnp
from jax import lax
from jax.experimental import pallas as pl

_IM = 1024.0
_PAD = 4.0
_MIN_SIZE = 16.0
_PRE_N = 6000
_POST_N = 300
_NMS_T = 0.7
_N = 12288
_ROWS = 96
_COLS = 128
_NEG = -1e30


def _body(c0_ref, c1_ref, d0_ref, d1_ref, d2_ref, d3_ref, d4_ref,
          ax1_ref, ay1_ref, ax2_ref, ay2_ref, out_ref):
    shape = (_ROWS, _COLS)
    c0 = c0_ref[...]
    c1 = c1_ref[...]
    d0 = d0_ref[...]
    d1 = d1_ref[...]
    d2 = d2_ref[...]
    d3 = d3_ref[...]
    d4 = d4_ref[...]
    ax1 = ax1_ref[...]
    ay1 = ay1_ref[...]
    ax2 = ax2_ref[...]
    ay2 = ay2_ref[...]
    m = jnp.maximum(c0, c1)
    e0 = jnp.exp(c0 - m)
    e1 = jnp.exp(c1 - m)
    score = e1 / (e0 + e1)
    widths = ax2 - ax1 + 1.0
    heights = ay2 - ay1 + 1.0
    ctr_x = ax1 + 0.5 * widths
    ctr_y = ay1 + 0.5 * heights
    cx = d0 * widths + ctr_x
    cy = d1 * heights + ctr_y
    a = jnp.exp(d2) * widths * 0.5
    b = jnp.exp(d3) * heights * 0.5
    th = d4
    ct = jnp.cos(th)
    st = jnp.sin(th)
    hw = jnp.sqrt((a * ct) ** 2 + (b * st) ** 2) + _PAD
    hh = jnp.sqrt((a * st) ** 2 + (b * ct) ** 2) + _PAD
    x1 = jnp.clip(cx - hw, 0.0, _IM - 1.0)
    y1 = jnp.clip(cy - hh, 0.0, _IM - 1.0)
    x2 = jnp.clip(cx + hw, 0.0, _IM - 1.0)
    y2 = jnp.clip(cy + hh, 0.0, _IM - 1.0)
    ws = x2 - x1 + 1.0
    hs = y2 - y1 + 1.0
    valid = (ws >= _MIN_SIZE) & (hs >= _MIN_SIZE)
    score = jnp.where(valid, score, jnp.float32(-1e9))
    areas = ws * hs

    u = lax.bitcast_convert_type(score, jnp.int32)
    ordv = u ^ (lax.shift_right_arithmetic(u, 31) & jnp.int32(0x7FFFFFFF))
    lin = (lax.broadcasted_iota(jnp.int32, shape, 0) * _COLS
           + lax.broadcasted_iota(jnp.int32, shape, 1))

    def _bs1(_, lohi):
        lo, hi = lohi
        mid = (lo & hi) + ((lo ^ hi) >> 1)
        ge = jnp.sum((ordv >= mid).astype(jnp.int32)) >= _PRE_N
        return (jnp.where(ge, mid, lo), jnp.where(ge, hi, mid))

    tau, _ = lax.fori_loop(
        0, 32, _bs1, (jnp.int32(-2147483647 - 1), jnp.int32(2147483647)))

    n_greater = jnp.sum((ordv > tau).astype(jnp.int32))
    quota = _PRE_N - n_greater
    tie = ordv == tau

    def _bs2(_, lohi):
        lo, hi = lohi
        mid = (lo + hi) >> 1
        ge = jnp.sum((tie & (lin <= mid)).astype(jnp.int32)) >= quota
        return (jnp.where(ge, lo, mid), jnp.where(ge, mid, hi))

    _, idxcut = lax.fori_loop(0, 14, _bs2, (jnp.int32(-1), jnp.int32(_N - 1)))

    eligible = (ordv > tau) | (tie & (lin <= idxcut))
    ninf = jnp.float32(-jnp.inf)
    live0 = jnp.where(eligible, score, ninf)
    lane = lax.broadcasted_iota(jnp.int32, (1, _COLS), 1)

    def _loop(i, carry):
        live, frow = carry
        mx = jnp.max(live)
        sel = jnp.min(jnp.where(live == mx, lin, jnp.int32(0x7FFFFFFF)))
        pickmask = lin == sel

        def pick(v):
            return jnp.max(jnp.where(pickmask, v, jnp.float32(_NEG)))

        sx1 = pick(x1)
        sy1 = pick(y1)
        sx2 = pick(x2)
        sy2 = pick(y2)
        scx = pick(cx)
        scy = pick(cy)
        sa = pick(a)
        sb = pick(b)
        sth = pick(th)
        ssc = pick(score)
        sarea = pick(areas)
        xx1 = jnp.maximum(x1, sx1)
        yy1 = jnp.maximum(y1, sy1)
        xx2 = jnp.minimum(x2, sx2)
        yy2 = jnp.minimum(y2, sy2)
        inter = (jnp.maximum(xx2 - xx1 + 1.0, 0.0)
                 * jnp.maximum(yy2 - yy1 + 1.0, 0.0))
        iou = inter / (areas + sarea - inter)
        live = jnp.where(iou > _NMS_T, ninf, live)
        row = jnp.zeros((1, _COLS), jnp.float32)
        for k, v in enumerate((sx1, sy1, sx2, sy2, scx, scy, sa, sb, sth, ssc)):
            row = jnp.where(lane == k, v, row)
        frow = jnp.where(i == 0, row, frow)
        rowf = jnp.where(mx == ninf, frow, row)
        out_ref[pl.ds(i, 1), :] = rowf
        return live, frow

    lax.fori_loop(0, _POST_N, _loop,
                  (live0, jnp.zeros((1, _COLS), jnp.float32)))


def kernel(out_cls, out_ellipse, anchors):
    c0 = out_cls[..., 0].reshape(_ROWS, _COLS)
    c1 = out_cls[..., 1].reshape(_ROWS, _COLS)
    ds = [out_ellipse[..., i].reshape(_ROWS, _COLS) for i in range(5)]
    axs = [anchors[:, i].reshape(_ROWS, _COLS) for i in range(4)]
    out = pl.pallas_call(
        _body,
        out_shape=jax.ShapeDtypeStruct((_POST_N, _COLS), jnp.float32),
    )(c0, c1, *ds, *axs)
    boxes = out[:, 0:4]
    ellipses = out[:, 4:9]
    scores = out[:, 9]
    return boxes, ellipses, scores


# speedup vs baseline: 15.9737x; 1.0488x over previous
import jax
import jax.numpy as jnp
from jax import lax
from jax.experimental import pallas as pl
from jax.experimental.pallas import tpu as pltpu

_IM = 1024.0
_PAD = 4.0
_MIN_SIZE = 16.0
_PRE_N = 6000
_POST_N = 300
_NMS_T = 0.7
_N = 12288
_ROWS = 96
_COLS = 128
_NEG = -1e30


def _body(c0_ref, c1_ref, d0_ref, d1_ref, d2_ref, d3_ref, d4_ref,
          ax1_ref, ay1_ref, ax2_ref, ay2_ref, out_ref, par_ref):
    shape = (_ROWS, _COLS)
    c0 = c0_ref[...]
    c1 = c1_ref[...]
    d0 = d0_ref[...]
    d1 = d1_ref[...]
    d2 = d2_ref[...]
    d3 = d3_ref[...]
    d4 = d4_ref[...]
    ax1 = ax1_ref[...]
    ay1 = ay1_ref[...]
    ax2 = ax2_ref[...]
    ay2 = ay2_ref[...]
    m = jnp.maximum(c0, c1)
    e0 = jnp.exp(c0 - m)
    e1 = jnp.exp(c1 - m)
    score = e1 / (e0 + e1)
    widths = ax2 - ax1 + 1.0
    heights = ay2 - ay1 + 1.0
    ctr_x = ax1 + 0.5 * widths
    ctr_y = ay1 + 0.5 * heights
    cx = d0 * widths + ctr_x
    cy = d1 * heights + ctr_y
    a = jnp.exp(d2) * widths * 0.5
    b = jnp.exp(d3) * heights * 0.5
    th = d4
    ct = jnp.cos(th)
    st = jnp.sin(th)
    hw = jnp.sqrt((a * ct) ** 2 + (b * st) ** 2) + _PAD
    hh = jnp.sqrt((a * st) ** 2 + (b * ct) ** 2) + _PAD
    x1 = jnp.clip(cx - hw, 0.0, _IM - 1.0)
    y1 = jnp.clip(cy - hh, 0.0, _IM - 1.0)
    x2 = jnp.clip(cx + hw, 0.0, _IM - 1.0)
    y2 = jnp.clip(cy + hh, 0.0, _IM - 1.0)
    ws = x2 - x1 + 1.0
    hs = y2 - y1 + 1.0
    valid = (ws >= _MIN_SIZE) & (hs >= _MIN_SIZE)
    score = jnp.where(valid, score, jnp.float32(-1e9))
    areas = ws * hs

    u = lax.bitcast_convert_type(score, jnp.int32)
    ordv = u ^ (lax.shift_right_arithmetic(u, 31) & jnp.int32(0x7FFFFFFF))
    lin = (lax.broadcasted_iota(jnp.int32, shape, 0) * _COLS
           + lax.broadcasted_iota(jnp.int32, shape, 1))

    def _bs1(_, lohi):
        lo, hi = lohi
        mid = (lo & hi) + ((lo ^ hi) >> 1)
        ge = jnp.sum((ordv >= mid).astype(jnp.int32)) >= _PRE_N
        return (jnp.where(ge, mid, lo), jnp.where(ge, hi, mid))

    tau, _ = lax.fori_loop(
        0, 32, _bs1, (jnp.int32(-2147483647 - 1), jnp.int32(2147483647)))

    n_greater = jnp.sum((ordv > tau).astype(jnp.int32))
    quota = _PRE_N - n_greater
    tie = ordv == tau

    def _bs2(_, lohi):
        lo, hi = lohi
        mid = (lo + hi) >> 1
        ge = jnp.sum((tie & (lin <= mid)).astype(jnp.int32)) >= quota
        return (jnp.where(ge, lo, mid), jnp.where(ge, mid, hi))

    _, idxcut = lax.fori_loop(0, 14, _bs2, (jnp.int32(-1), jnp.int32(_N - 1)))

    eligible = (ordv > tau) | (tie & (lin <= idxcut))
    ninf = jnp.float32(-jnp.inf)
    live0 = jnp.where(eligible, score, ninf)
    lane = lax.broadcasted_iota(jnp.int32, (1, _COLS), 1)

    for k, v in enumerate((x1, y1, x2, y2, cx, cy, a, b, th, score, areas)):
        par_ref[k, :, :] = v

    def _loop(i, carry):
        live, frow = carry
        mx = jnp.max(live)
        sel = jnp.min(jnp.where(live == mx, lin, jnp.int32(0x7FFFFFFF)))
        r = sel >> 7
        c = sel & 127
        lanemask = lane == c

        def pick(k):
            return jnp.max(jnp.where(lanemask, par_ref[k, pl.ds(r, 1), :],
                                     jnp.float32(_NEG)))

        sx1 = pick(0)
        sy1 = pick(1)
        sx2 = pick(2)
        sy2 = pick(3)
        scx = pick(4)
        scy = pick(5)
        sa = pick(6)
        sb = pick(7)
        sth = pick(8)
        ssc = pick(9)
        sarea = pick(10)
        xx1 = jnp.maximum(x1, sx1)
        yy1 = jnp.maximum(y1, sy1)
        xx2 = jnp.minimum(x2, sx2)
        yy2 = jnp.minimum(y2, sy2)
        inter = (jnp.maximum(xx2 - xx1 + 1.0, 0.0)
                 * jnp.maximum(yy2 - yy1 + 1.0, 0.0))
        iou = inter / (areas + sarea - inter)
        live = jnp.where(iou > _NMS_T, ninf, live)
        row = jnp.zeros((1, _COLS), jnp.float32)
        for k, v in enumerate((sx1, sy1, sx2, sy2, scx, scy, sa, sb, sth, ssc)):
            row = jnp.where(lane == k, v, row)
        frow = jnp.where(i == 0, row, frow)
        rowf = jnp.where(mx == ninf, frow, row)
        out_ref[pl.ds(i, 1), :] = rowf
        return live, frow

    lax.fori_loop(0, _POST_N, _loop,
                  (live0, jnp.zeros((1, _COLS), jnp.float32)))


def kernel(out_cls, out_ellipse, anchors):
    c0 = out_cls[..., 0].reshape(_ROWS, _COLS)
    c1 = out_cls[..., 1].reshape(_ROWS, _COLS)
    ds = [out_ellipse[..., i].reshape(_ROWS, _COLS) for i in range(5)]
    axs = [anchors[:, i].reshape(_ROWS, _COLS) for i in range(4)]
    out = pl.pallas_call(
        _body,
        out_shape=jax.ShapeDtypeStruct((_POST_N, _COLS), jnp.float32),
        scratch_shapes=[pltpu.VMEM((11, _ROWS, _COLS), jnp.float32)],
    )(c0, c1, *ds, *axs)
    boxes = out[:, 0:4]
    ellipses = out[:, 4:9]
    scores = out[:, 9]
    return boxes, ellipses, scores
